# Initial kernel scaffold; baseline (speedup 1.0000x reference)
import jax
import jax.numpy as jnp
from jax.experimental import pallas as pl


def kernel(x, edge_index, batch, W0l, W0r, b0, W1l, W1r, b1, Wlin, blin):
    raise NotImplementedError("write your pallas kernel here")


# speedup vs baseline: 4.1758x; 4.1758x over previous
import functools

import jax
import jax.numpy as jnp
from jax import lax
from jax.experimental import pallas as pl
from jax.experimental.pallas import tpu as pltpu
from jax.experimental.pallas import tpu_sc as plsc

NC = 2
NS = 16
NW = NC * NS
CHUNK = 128


def _make_sc_agg(n_pad, d, e_pad, with_deg):
  t_steps = e_pad // (NW * CHUNK)
  rowch = n_pad // CHUNK
  mesh = plsc.VectorSubcoreMesh(
      core_axis_name="c", subcore_axis_name="s",
      num_cores=NC, num_subcores=NS)

  out_type = [jax.ShapeDtypeStruct((NC, n_pad, d), jnp.float32)]
  if with_deg:
    out_type.append(jax.ShapeDtypeStruct((NC, n_pad), jnp.float32))

  scratch = [
      pltpu.VMEM((CHUNK,), jnp.int32),
      pltpu.VMEM((CHUNK,), jnp.int32),
      pltpu.VMEM((CHUNK, d), jnp.float32),
      pltpu.VMEM((CHUNK, d), jnp.float32),
      pltpu.VMEM((CHUNK,), jnp.float32),
      pltpu.VMEM_SHARED((n_pad, d), jnp.float32),
      pltpu.VMEM_SHARED((n_pad,), jnp.float32),
      pltpu.SemaphoreType.DMA,
  ]

  def body(feats, srci, dsti, z2d, ones_in, *rest):
    if with_deg:
      acc_out, deg_out = rest[0], rest[1]
      sidx, didx, rows, zrow, ones_v, acc_sh, deg_sh, sem = rest[2:]
    else:
      acc_out = rest[0]
      deg_out = None
      sidx, didx, rows, zrow, ones_v, acc_sh, deg_sh, sem = rest[1:]

    c = lax.axis_index("c")
    s = lax.axis_index("s")
    wid = s * NC + c

    pltpu.sync_copy(z2d, zrow)
    pltpu.sync_copy(ones_in, ones_v)

    for jj in range((rowch + NS - 1) // NS):
      j = jj * NS + s

      @pl.when(j < rowch)
      def _():
        pltpu.sync_copy(zrow, acc_sh.at[pl.ds(j * CHUNK, CHUNK)])
        if with_deg:
          pltpu.sync_copy(zrow.at[0], deg_sh.at[pl.ds(j * CHUNK, CHUNK)])

    plsc.subcore_barrier()

    base0 = wid * t_steps * CHUNK

    def step(t, carry):
      base = base0 + t * CHUNK
      pltpu.sync_copy(srci.at[pl.ds(base, CHUNK)], sidx)
      pltpu.sync_copy(dsti.at[pl.ds(base, CHUNK)], didx)
      pltpu.async_copy(feats.at[sidx], rows, sem).wait()
      pltpu.sync_copy(rows, acc_sh.at[didx], add=True)
      if with_deg:
        pltpu.sync_copy(ones_v, deg_sh.at[didx], add=True)
      return carry

    lax.fori_loop(0, t_steps, step, 0)

    plsc.subcore_barrier()

    for jj in range((rowch + NS - 1) // NS):
      j = jj * NS + s

      @pl.when(j < rowch)
      def _():
        pltpu.sync_copy(acc_sh.at[pl.ds(j * CHUNK, CHUNK)],
                        acc_out.at[c, pl.ds(j * CHUNK, CHUNK)])
        if with_deg:
          pltpu.sync_copy(deg_sh.at[pl.ds(j * CHUNK, CHUNK)],
                          deg_out.at[c, pl.ds(j * CHUNK, CHUNK)])

  return pl.kernel(body, out_type=out_type, mesh=mesh, scratch_types=scratch)


def _tc_layer(acc, deg3, xin, wl, wr, b, row_blk):
  n, d = xin.shape
  nb = n // row_blk

  def body(acc_ref, deg_ref, x_ref, wl_ref, wr_ref, b_ref, o_ref):
    dsl = deg_ref[0]
    dtot = jnp.maximum(dsl[0] + dsl[1], 1.0)
    agg = (acc_ref[0] + acc_ref[1]) / dtot[:, None]
    h = (jnp.dot(agg, wl_ref[...], preferred_element_type=jnp.float32)
         + jnp.dot(x_ref[...], wr_ref[...], preferred_element_type=jnp.float32)
         + b_ref[...])
    o_ref[...] = jnp.maximum(h, 0.0)

  return pl.pallas_call(
      body,
      grid=(nb,),
      in_specs=[
          pl.BlockSpec((NC, row_blk, d), lambda i: (0, i, 0)),
          pl.BlockSpec((1, NC, row_blk), lambda i: (i, 0, 0)),
          pl.BlockSpec((row_blk, d), lambda i: (i, 0)),
          pl.BlockSpec((d, d), lambda i: (0, 0)),
          pl.BlockSpec((d, d), lambda i: (0, 0)),
          pl.BlockSpec((1, d), lambda i: (0, 0)),
      ],
      out_specs=pl.BlockSpec((row_blk, d), lambda i: (i, 0)),
      out_shape=jax.ShapeDtypeStruct((n, d), jnp.float32),
  )(acc, deg3, xin, wl, wr, b)


def _tc_final(acc, deg3, h1, wl, wr, b, bat3, wlin_p, blin_p, n_graphs,
              row_blk):
  n, d = h1.shape
  nb = n // row_blk

  def body(acc_ref, deg_ref, h1_ref, wl_ref, wr_ref, b_ref, bat_ref,
           wlin_ref, blin_ref, o_ref, pool_s, cnt_s):
    i = pl.program_id(0)
    dsl = deg_ref[0]
    dtot = jnp.maximum(dsl[0] + dsl[1], 1.0)
    agg = (acc_ref[0] + acc_ref[1]) / dtot[:, None]
    h2 = jnp.maximum(
        jnp.dot(agg, wl_ref[...], preferred_element_type=jnp.float32)
        + jnp.dot(h1_ref[...], wr_ref[...], preferred_element_type=jnp.float32)
        + b_ref[...], 0.0)

    bat = bat_ref[0, 0, :]
    seg = lax.broadcasted_iota(jnp.int32, (n_graphs, row_blk), 0)
    m = (seg == bat[None, :]).astype(jnp.float32)
    p_part = jnp.dot(m, h2, preferred_element_type=jnp.float32)
    c_part = jnp.broadcast_to(
        jnp.sum(m, axis=1, keepdims=True), (n_graphs, d))

    @pl.when(i == 0)
    def _():
      pool_s[...] = p_part
      cnt_s[...] = c_part

    @pl.when(i > 0)
    def _():
      pool_s[...] = pool_s[...] + p_part
      cnt_s[...] = cnt_s[...] + c_part

    @pl.when(i == nb - 1)
    def _():
      pooled = pool_s[...] / jnp.maximum(cnt_s[...], 1.0)
      o_ref[...] = (jnp.dot(pooled, wlin_ref[...],
                            preferred_element_type=jnp.float32)
                    + blin_ref[...])

  return pl.pallas_call(
      body,
      grid=(nb,),
      in_specs=[
          pl.BlockSpec((NC, row_blk, d), lambda i: (0, i, 0)),
          pl.BlockSpec((1, NC, row_blk), lambda i: (i, 0, 0)),
          pl.BlockSpec((row_blk, d), lambda i: (i, 0)),
          pl.BlockSpec((d, d), lambda i: (0, 0)),
          pl.BlockSpec((d, d), lambda i: (0, 0)),
          pl.BlockSpec((1, d), lambda i: (0, 0)),
          pl.BlockSpec((1, 1, row_blk), lambda i: (i, 0, 0)),
          pl.BlockSpec((d, d), lambda i: (0, 0)),
          pl.BlockSpec((1, d), lambda i: (0, 0)),
      ],
      out_specs=pl.BlockSpec((n_graphs, d), lambda i: (0, 0)),
      out_shape=jax.ShapeDtypeStruct((n_graphs, d), jnp.float32),
      scratch_shapes=[
          pltpu.VMEM((n_graphs, d), jnp.float32),
          pltpu.VMEM((n_graphs, d), jnp.float32),
      ],
  )(acc, deg3, h1, wl, wr, b, bat3, wlin_p, blin_p)


def kernel(x, edge_index, batch, W0l, W0r, b0, W1l, W1r, b1, Wlin, blin):
  n, d = x.shape
  e = edge_index.shape[1]
  n_cls = Wlin.shape[1]
  n_graphs = 64
  row_blk = 400
  nb = n // row_blk

  grain = NW * CHUNK
  e_pad = ((e + grain - 1) // grain) * grain
  n_pad = ((n + 1 + CHUNK - 1) // CHUNK) * CHUNK

  src = edge_index[0]
  dst = edge_index[1]
  pad = e_pad - e
  if pad:
    src = jnp.concatenate([src, jnp.zeros((pad,), jnp.int32)])
    dst = jnp.concatenate([dst, jnp.full((pad,), n, jnp.int32)])
  z2d = jnp.zeros((CHUNK, d), jnp.float32)
  ones1 = jnp.ones((CHUNK,), jnp.float32)

  acc0, deg = _make_sc_agg(n_pad, d, e_pad, True)(x, src, dst, z2d, ones1)
  deg3 = deg[:, :n].reshape(NC, nb, row_blk).transpose(1, 0, 2)

  b0r = b0.reshape(1, d)
  b1r = b1.reshape(1, d)
  h1 = _tc_layer(acc0, deg3, x, W0l, W0r, b0r, row_blk)

  (acc1,) = _make_sc_agg(n_pad, d, e_pad, False)(h1, src, dst, z2d, ones1)

  bat3 = batch.reshape(nb, 1, row_blk)
  wlin_p = jnp.zeros((d, d), jnp.float32).at[:, :n_cls].set(Wlin)
  blin_p = jnp.zeros((1, d), jnp.float32).at[0, :n_cls].set(blin)

  logits = _tc_final(acc1, deg3, h1, W1l, W1r, b1r, bat3, wlin_p, blin_p,
                     n_graphs, row_blk)
  return logits[:, :n_cls]
